# baseline (speedup 1.0000x reference)
import jax
import jax.numpy as jnp
from jax.experimental import pallas as pl

_K = 1024
_D = 64
_BETA = 0.25
_TB = 512


def _vq_block(lat_ref, w_ref, q_ref, idx_ref, acc_ref):
    b = pl.program_id(0)
    t = pl.program_id(1)

    x = lat_ref[0]
    w = w_ref[...]
    xt = x.T

    x2 = jnp.sum(xt * xt, axis=1, keepdims=True)
    w2 = jnp.sum(w * w, axis=1)
    s = jax.lax.dot_general(
        xt, w, (((1,), (1,)), ((), ())),
        preferred_element_type=jnp.float32,
        precision=jax.lax.Precision.DEFAULT)
    dist = (x2 + w2[None, :]) - 2.0 * s

    m = jnp.min(dist, axis=1, keepdims=True)
    iota = jax.lax.broadcasted_iota(jnp.int32, dist.shape, 1)
    idx = jnp.min(jnp.where(dist == m, iota, _K), axis=1)
    idx_ref[0, 0, :] = idx

    onehot = (iota == idx[:, None]).astype(jnp.float32)
    q = jax.lax.dot_general(
        w, onehot, (((0,), (1,)), ((), ())),
        preferred_element_type=jnp.float32,
        precision=jax.lax.Precision.HIGHEST)
    q_ref[0] = q

    diff = q - x
    part = jnp.sum(diff * diff, axis=0)

    @pl.when(jnp.logical_and(b == 0, t == 0))
    def _():
        acc_ref[0, :] = jnp.zeros((_TB,), jnp.float32)

    acc_ref[0, :] += part


def kernel(latents, weight):
    B, D, T = latents.shape
    nt = T // _TB
    q, idx3, acc = pl.pallas_call(
        _vq_block,
        grid=(B, nt),
        in_specs=[
            pl.BlockSpec((1, D, _TB), lambda b, t: (b, 0, t)),
            pl.BlockSpec((_K, _D), lambda b, t: (0, 0)),
        ],
        out_specs=[
            pl.BlockSpec((1, D, _TB), lambda b, t: (b, 0, t)),
            pl.BlockSpec((1, 1, _TB), lambda b, t: (b, 0, t)),
            pl.BlockSpec((1, _TB), lambda b, t: (0, 0)),
        ],
        out_shape=[
            jax.ShapeDtypeStruct((B, D, T), jnp.float32),
            jax.ShapeDtypeStruct((B, 1, T), jnp.int32),
            jax.ShapeDtypeStruct((1, _TB), jnp.float32),
        ],
    )(latents, weight)
    mean_sq = jnp.sum(acc) / (B * T * D)
    loss = mean_sq + _BETA * mean_sq
    return q, loss, idx3.reshape(B, T)


# speedup vs baseline: 1.8950x; 1.8950x over previous
import jax
import jax.numpy as jnp
from jax.experimental import pallas as pl

_K = 1024
_D = 64
_BETA = 0.25
_TB = 512


def _vq_block(lat_ref, w_ref, q_ref, idx_ref, acc_ref):
    b = pl.program_id(0)
    t = pl.program_id(1)

    x = lat_ref[0]
    w = w_ref[...]
    xt = x.T

    x2 = jnp.sum(xt * xt, axis=1, keepdims=True)
    w2 = jnp.sum(w * w, axis=1)
    s = jax.lax.dot_general(
        xt, w, (((1,), (1,)), ((), ())),
        preferred_element_type=jnp.float32,
        precision=jax.lax.Precision.DEFAULT)
    dist = (x2 + w2[None, :]) - 2.0 * s

    m = jnp.min(dist, axis=1, keepdims=True)
    iota = jax.lax.broadcasted_iota(jnp.int32, dist.shape, 1)
    idx = jnp.min(jnp.where(dist == m, iota, _K), axis=1)
    idx_ref[0, 0, :] = idx

    onehot = (iota == idx[:, None]).astype(jnp.bfloat16)
    q = jax.lax.dot_general(
        w.astype(jnp.bfloat16), onehot, (((0,), (1,)), ((), ())),
        preferred_element_type=jnp.float32)
    q_ref[0] = q

    @pl.when(jnp.logical_and(b == 0, t == 0))
    def _():
        acc_ref[...] = jnp.zeros((_TB, 1), jnp.float32)

    acc_ref[...] += m


def kernel(latents, weight):
    B, D, T = latents.shape
    nt = T // _TB
    q, idx3, acc = pl.pallas_call(
        _vq_block,
        grid=(B, nt),
        in_specs=[
            pl.BlockSpec((1, D, _TB), lambda b, t: (b, 0, t)),
            pl.BlockSpec((_K, _D), lambda b, t: (0, 0)),
        ],
        out_specs=[
            pl.BlockSpec((1, D, _TB), lambda b, t: (b, 0, t)),
            pl.BlockSpec((1, 1, _TB), lambda b, t: (b, 0, t)),
            pl.BlockSpec((_TB, 1), lambda b, t: (0, 0)),
        ],
        out_shape=[
            jax.ShapeDtypeStruct((B, D, T), jnp.float32),
            jax.ShapeDtypeStruct((B, 1, T), jnp.int32),
            jax.ShapeDtypeStruct((_TB, 1), jnp.float32),
        ],
    )(latents, weight)
    mean_sq = jnp.sum(acc) / (B * T * D)
    loss = mean_sq + _BETA * mean_sq
    return q, loss, idx3.reshape(B, T)


# speedup vs baseline: 2.0922x; 1.1040x over previous
import jax
import jax.numpy as jnp
from jax.experimental import pallas as pl
from jax.experimental.pallas import tpu as pltpu

_K = 1024
_D = 64
_BETA = 0.25
_TB = 512


def _vq_block(lat_ref, w_ref, q_ref, idx_ref, acc_ref, w2_ref, wbf_ref):
    b = pl.program_id(0)
    t = pl.program_id(1)

    @pl.when(jnp.logical_and(b == 0, t == 0))
    def _():
        w0 = w_ref[...]
        w2_ref[...] = jnp.sum(w0 * w0, axis=1)[None, :]
        wbf_ref[...] = w0.astype(jnp.bfloat16)
        acc_ref[...] = jnp.zeros((_TB, 1), jnp.float32)

    x = lat_ref[0]
    xt = x.T

    x2 = jnp.sum(xt * xt, axis=1, keepdims=True)
    s = jax.lax.dot_general(
        xt, w_ref[...], (((1,), (1,)), ((), ())),
        preferred_element_type=jnp.float32,
        precision=jax.lax.Precision.DEFAULT)
    dist = (x2 + w2_ref[...]) - 2.0 * s

    m = jnp.min(dist, axis=1, keepdims=True)
    iota = jax.lax.broadcasted_iota(jnp.int32, dist.shape, 1)
    idx = jnp.min(jnp.where(dist == m, iota, _K), axis=1)
    idx_ref[0, 0, :] = idx

    onehot = (iota == idx[:, None]).astype(jnp.bfloat16)
    q = jax.lax.dot_general(
        wbf_ref[...], onehot, (((0,), (1,)), ((), ())),
        preferred_element_type=jnp.float32)
    q_ref[0] = q

    acc_ref[...] += m


def kernel(latents, weight):
    B, D, T = latents.shape
    nt = T // _TB
    q, idx3, acc = pl.pallas_call(
        _vq_block,
        grid=(B, nt),
        in_specs=[
            pl.BlockSpec((1, D, _TB), lambda b, t: (b, 0, t)),
            pl.BlockSpec((_K, _D), lambda b, t: (0, 0)),
        ],
        out_specs=[
            pl.BlockSpec((1, D, _TB), lambda b, t: (b, 0, t)),
            pl.BlockSpec((1, 1, _TB), lambda b, t: (b, 0, t)),
            pl.BlockSpec((_TB, 1), lambda b, t: (0, 0)),
        ],
        out_shape=[
            jax.ShapeDtypeStruct((B, D, T), jnp.float32),
            jax.ShapeDtypeStruct((B, 1, T), jnp.int32),
            jax.ShapeDtypeStruct((_TB, 1), jnp.float32),
        ],
        scratch_shapes=[
            pltpu.VMEM((1, _K), jnp.float32),
            pltpu.VMEM((_K, _D), jnp.bfloat16),
        ],
    )(latents, weight)
    mean_sq = jnp.sum(acc) / (B * T * D)
    loss = mean_sq + _BETA * mean_sq
    return q, loss, idx3.reshape(B, T)


# speedup vs baseline: 2.2372x; 1.0693x over previous
import jax
import jax.numpy as jnp
from jax.experimental import pallas as pl
from jax.experimental.pallas import tpu as pltpu

_K = 1024
_D = 64
_BETA = 0.25
_TB = 1024


def _vq_block(lat_ref, w_ref, q_ref, idx_ref, acc_ref, w2_ref, wbf_ref):
    b = pl.program_id(0)
    t = pl.program_id(1)

    @pl.when(jnp.logical_and(b == 0, t == 0))
    def _():
        w0 = w_ref[...]
        w2_ref[...] = jnp.sum(w0 * w0, axis=1)[None, :]
        wbf_ref[...] = w0.astype(jnp.bfloat16)
        acc_ref[...] = jnp.zeros((_TB, 1), jnp.float32)

    x = lat_ref[0]
    xt = x.T

    x2 = jnp.sum(xt * xt, axis=1, keepdims=True)
    s = jax.lax.dot_general(
        xt, w_ref[...], (((1,), (1,)), ((), ())),
        preferred_element_type=jnp.float32,
        precision=jax.lax.Precision.DEFAULT)
    dist = (x2 + w2_ref[...]) - 2.0 * s

    m = jnp.min(dist, axis=1, keepdims=True)
    iota = jax.lax.broadcasted_iota(jnp.int32, dist.shape, 1)
    idx = jnp.min(jnp.where(dist == m, iota, _K), axis=1)
    idx_ref[0, 0, :] = idx

    onehot = (iota == idx[:, None]).astype(jnp.bfloat16)
    q = jax.lax.dot_general(
        wbf_ref[...], onehot, (((0,), (1,)), ((), ())),
        preferred_element_type=jnp.float32)
    q_ref[0] = q

    acc_ref[...] += m


def kernel(latents, weight):
    B, D, T = latents.shape
    nt = T // _TB
    q, idx3, acc = pl.pallas_call(
        _vq_block,
        grid=(B, nt),
        in_specs=[
            pl.BlockSpec((1, D, _TB), lambda b, t: (b, 0, t)),
            pl.BlockSpec((_K, _D), lambda b, t: (0, 0)),
        ],
        out_specs=[
            pl.BlockSpec((1, D, _TB), lambda b, t: (b, 0, t)),
            pl.BlockSpec((1, 1, _TB), lambda b, t: (b, 0, t)),
            pl.BlockSpec((_TB, 1), lambda b, t: (0, 0)),
        ],
        out_shape=[
            jax.ShapeDtypeStruct((B, D, T), jnp.float32),
            jax.ShapeDtypeStruct((B, 1, T), jnp.int32),
            jax.ShapeDtypeStruct((_TB, 1), jnp.float32),
        ],
        scratch_shapes=[
            pltpu.VMEM((1, _K), jnp.float32),
            pltpu.VMEM((_K, _D), jnp.bfloat16),
        ],
    )(latents, weight)
    mean_sq = jnp.sum(acc) / (B * T * D)
    loss = mean_sq + _BETA * mean_sq
    return q, loss, idx3.reshape(B, T)


# speedup vs baseline: 2.4748x; 1.1062x over previous
import jax
import jax.numpy as jnp
from jax.experimental import pallas as pl
from jax.experimental.pallas import tpu as pltpu

_K = 1024
_D = 64
_BETA = 0.25
_TB = 2048


def _vq_block(lat_ref, w_ref, q_ref, idx_ref, acc_ref, w2_ref, wbf_ref):
    b = pl.program_id(0)
    t = pl.program_id(1)

    @pl.when(jnp.logical_and(b == 0, t == 0))
    def _():
        w0 = w_ref[...]
        w2_ref[...] = jnp.sum(w0 * w0, axis=1)[None, :]
        wbf_ref[...] = w0.astype(jnp.bfloat16)
        acc_ref[...] = jnp.zeros((_TB, 1), jnp.float32)

    x = lat_ref[0]
    xt = x.T

    x2 = jnp.sum(xt * xt, axis=1, keepdims=True)
    s = jax.lax.dot_general(
        xt, w_ref[...], (((1,), (1,)), ((), ())),
        preferred_element_type=jnp.float32,
        precision=jax.lax.Precision.DEFAULT)
    dist = (x2 + w2_ref[...]) - 2.0 * s

    m = jnp.min(dist, axis=1, keepdims=True)
    iota = jax.lax.broadcasted_iota(jnp.int32, dist.shape, 1)
    idx = jnp.min(jnp.where(dist == m, iota, _K), axis=1)
    idx_ref[0, 0, :] = idx

    onehot = (iota == idx[:, None]).astype(jnp.bfloat16)
    q = jax.lax.dot_general(
        wbf_ref[...], onehot, (((0,), (1,)), ((), ())),
        preferred_element_type=jnp.float32)
    q_ref[0] = q

    acc_ref[...] += m


def kernel(latents, weight):
    B, D, T = latents.shape
    nt = T // _TB
    q, idx3, acc = pl.pallas_call(
        _vq_block,
        grid=(B, nt),
        in_specs=[
            pl.BlockSpec((1, D, _TB), lambda b, t: (b, 0, t)),
            pl.BlockSpec((_K, _D), lambda b, t: (0, 0)),
        ],
        out_specs=[
            pl.BlockSpec((1, D, _TB), lambda b, t: (b, 0, t)),
            pl.BlockSpec((1, 1, _TB), lambda b, t: (b, 0, t)),
            pl.BlockSpec((_TB, 1), lambda b, t: (0, 0)),
        ],
        out_shape=[
            jax.ShapeDtypeStruct((B, D, T), jnp.float32),
            jax.ShapeDtypeStruct((B, 1, T), jnp.int32),
            jax.ShapeDtypeStruct((_TB, 1), jnp.float32),
        ],
        scratch_shapes=[
            pltpu.VMEM((1, _K), jnp.float32),
            pltpu.VMEM((_K, _D), jnp.bfloat16),
        ],
    )(latents, weight)
    mean_sq = jnp.sum(acc) / (B * T * D)
    loss = mean_sq + _BETA * mean_sq
    return q, loss, idx3.reshape(B, T)
